# Initial kernel scaffold; baseline (speedup 1.0000x reference)
import jax
import jax.numpy as jnp
from jax.experimental import pallas as pl


def kernel(h, edge_index, adj, W_gat, attn_l, attn_r, b_gat, W_gcn, b_gcn):
    raise NotImplementedError("write your pallas kernel here")


# speedup vs baseline: 1.9028x; 1.9028x over previous
import functools

import jax
import jax.numpy as jnp
from jax.experimental import pallas as pl
from jax.experimental.pallas import tpu as pltpu

N = 8192
E = 262144
IN = 128
OUT = 64

ROW_BLK = 256
K_BLK = 2048


def _stage0_body(h_ref, wg_ref, al_ref, ar_ref, wc_ref, z_ref, el_ref, er_ref, sup_ref):
    h = h_ref[...]
    z = jnp.dot(h, wg_ref[...], preferred_element_type=jnp.float32)
    z_ref[...] = z
    el_ref[...] = jnp.sum(z * al_ref[...], axis=1, keepdims=True)
    er_ref[...] = jnp.sum(z * ar_ref[...], axis=1, keepdims=True)
    sup_ref[...] = jnp.dot(h, wc_ref[...], preferred_element_type=jnp.float32)


def _stage0(h, W_gat, attn_l, attn_r, W_gcn):
    grid = (N // ROW_BLK,)
    return pl.pallas_call(
        _stage0_body,
        grid=grid,
        in_specs=[
            pl.BlockSpec((ROW_BLK, IN), lambda i: (i, 0)),
            pl.BlockSpec((IN, OUT), lambda i: (0, 0)),
            pl.BlockSpec((1, OUT), lambda i: (0, 0)),
            pl.BlockSpec((1, OUT), lambda i: (0, 0)),
            pl.BlockSpec((IN, OUT), lambda i: (0, 0)),
        ],
        out_specs=[
            pl.BlockSpec((ROW_BLK, OUT), lambda i: (i, 0)),
            pl.BlockSpec((ROW_BLK, 1), lambda i: (i, 0)),
            pl.BlockSpec((ROW_BLK, 1), lambda i: (i, 0)),
            pl.BlockSpec((ROW_BLK, OUT), lambda i: (i, 0)),
        ],
        out_shape=[
            jax.ShapeDtypeStruct((N, OUT), jnp.float32),
            jax.ShapeDtypeStruct((N, 1), jnp.float32),
            jax.ShapeDtypeStruct((N, 1), jnp.float32),
            jax.ShapeDtypeStruct((N, OUT), jnp.float32),
        ],
    )(h, W_gat, attn_l.reshape(1, OUT), attn_r.reshape(1, OUT), W_gcn)


def _gcn_body(adj_ref, sup_ref, b_ref, out_ref):
    k = pl.program_id(1)

    @pl.when(k == 0)
    def _init():
        out_ref[...] = jnp.zeros_like(out_ref)

    out_ref[...] += jnp.dot(adj_ref[...], sup_ref[...],
                            preferred_element_type=jnp.float32)

    @pl.when(k == pl.num_programs(1) - 1)
    def _fini():
        x = out_ref[...] + b_ref[...]
        out_ref[...] = jnp.where(x > 0, x, jnp.exp(x) - 1.0)


def _gcn(adj, support, b_gcn):
    grid = (N // ROW_BLK, N // K_BLK)
    return pl.pallas_call(
        _gcn_body,
        grid=grid,
        in_specs=[
            pl.BlockSpec((ROW_BLK, K_BLK), lambda i, k: (i, k)),
            pl.BlockSpec((K_BLK, OUT), lambda i, k: (k, 0)),
            pl.BlockSpec((1, OUT), lambda i, k: (0, 0)),
        ],
        out_specs=pl.BlockSpec((ROW_BLK, OUT), lambda i, k: (i, 0)),
        out_shape=jax.ShapeDtypeStruct((N, OUT), jnp.float32),
        compiler_params=pltpu.CompilerParams(
            dimension_semantics=("parallel", "arbitrary"),
        ),
    )(adj, support, b_gcn.reshape(1, OUT))


def kernel(h, edge_index, adj, W_gat, attn_l, attn_r, b_gat, W_gcn, b_gcn):
    src = edge_index[0]
    dst = edge_index[1]
    z, el2, er2, support = _stage0(h, W_gat, attn_l, attn_r, W_gcn)
    el = el2[:, 0]
    er = er2[:, 0]

    m = jnp.maximum(jnp.max(el) + jnp.max(er), 0.0)
    e = el[src] + er[dst]
    e = jnp.where(e > 0, e, 0.2 * e)
    ex = jnp.exp(e - m)
    denom = jax.ops.segment_sum(ex, dst, num_segments=N)
    num = jax.ops.segment_sum(ex[:, None] * z[src], dst, num_segments=N)
    gat = num / jnp.maximum(denom, 1e-9)[:, None]
    gatb = gat + b_gat
    gat = jnp.where(gatb > 0, gatb, jnp.expm1(gatb))

    gcn = _gcn(adj, support, b_gcn)
    return jnp.concatenate([gat, gcn], axis=1)


# speedup vs baseline: 48.5027x; 25.4900x over previous
import functools

import jax
import jax.numpy as jnp
from jax import lax
from jax.experimental import pallas as pl
from jax.experimental.pallas import tpu as pltpu
from jax.experimental.pallas import tpu_sc as plsc

N = 8192
E = 262144
IN = 128
OUT = 64

NC = 2
NS = 16
L = 16
NW = NC * NS
EPW = E // NW
CH = 128
NCHUNK = EPW // CH
NSL = N // NS

ROW_BLK = 256
K_BLK = 2048


def _stage0_body(h_ref, wg_ref, al_ref, ar_ref, wc_ref,
                 z_ref, el_ref, er_ref, sup_ref, m_ref, sm_ref):
    i = pl.program_id(0)
    h = h_ref[...]
    z = jnp.dot(h, wg_ref[...], preferred_element_type=jnp.float32)
    z_ref[...] = z
    el = jnp.sum(z * al_ref[...], axis=1, keepdims=True)
    er = jnp.sum(z * ar_ref[...], axis=1, keepdims=True)
    el_ref[...] = el
    er_ref[...] = er
    sup_ref[...] = jnp.dot(h, wc_ref[...], preferred_element_type=jnp.float32)

    ml = jnp.max(el)
    mr = jnp.max(er)

    @pl.when(i == 0)
    def _init():
        sm_ref[0] = ml
        sm_ref[1] = mr

    @pl.when(i > 0)
    def _acc():
        sm_ref[0] = jnp.maximum(sm_ref[0], ml)
        sm_ref[1] = jnp.maximum(sm_ref[1], mr)

    @pl.when(i == pl.num_programs(0) - 1)
    def _fini():
        m_ref[...] = jnp.full((1, L), jnp.maximum(sm_ref[0] + sm_ref[1], 0.0))


def _stage0(h, W_gat, attn_l, attn_r, W_gcn):
    return pl.pallas_call(
        _stage0_body,
        grid=(N // ROW_BLK,),
        in_specs=[
            pl.BlockSpec((ROW_BLK, IN), lambda i: (i, 0)),
            pl.BlockSpec((IN, OUT), lambda i: (0, 0)),
            pl.BlockSpec((1, OUT), lambda i: (0, 0)),
            pl.BlockSpec((1, OUT), lambda i: (0, 0)),
            pl.BlockSpec((IN, OUT), lambda i: (0, 0)),
        ],
        out_specs=[
            pl.BlockSpec((ROW_BLK, OUT), lambda i: (i, 0)),
            pl.BlockSpec((ROW_BLK, 1), lambda i: (i, 0)),
            pl.BlockSpec((ROW_BLK, 1), lambda i: (i, 0)),
            pl.BlockSpec((ROW_BLK, OUT), lambda i: (i, 0)),
            pl.BlockSpec((1, L), lambda i: (0, 0)),
        ],
        out_shape=[
            jax.ShapeDtypeStruct((N, OUT), jnp.float32),
            jax.ShapeDtypeStruct((N, 1), jnp.float32),
            jax.ShapeDtypeStruct((N, 1), jnp.float32),
            jax.ShapeDtypeStruct((N, OUT), jnp.float32),
            jax.ShapeDtypeStruct((1, L), jnp.float32),
        ],
        scratch_shapes=[pltpu.SMEM((2,), jnp.float32)],
        compiler_params=pltpu.CompilerParams(
            dimension_semantics=("arbitrary",),
        ),
    )(h, W_gat, attn_l.reshape(1, OUT), attn_r.reshape(1, OUT), W_gcn)


def _edge_body(src_hbm, dst_hbm, el_hbm, er_hbm, m_hbm, z_hbm,
               denpp_hbm, gatp_hbm,
               src_v, dst_v, el_v, er_v, ex_v, den_v, m_v,
               rows0, rows1, zbuf, accsp, gsem):
    cid = lax.axis_index("c")
    sid = lax.axis_index("s")
    wid = sid * NC + cid

    pltpu.sync_copy(src_hbm.at[wid], src_v)
    pltpu.sync_copy(dst_hbm.at[wid], dst_v)
    pltpu.sync_copy(el_hbm, el_v)
    pltpu.sync_copy(er_hbm, er_v)
    pltpu.sync_copy(m_hbm, m_v)
    mvec = m_v[...]

    zero = jnp.zeros((L,), jnp.float32)

    def _zden(i, c):
        den_v[pl.ds(i * L, L)] = zero
        return c
    lax.fori_loop(0, N // L, _zden, 0)

    def _zbuf(r, c):
        for k in range(OUT // L):
            zbuf[r, pl.ds(k * L, L)] = zero
        return c
    lax.fori_loop(0, CH, _zbuf, 0)

    for q in range(NSL // CH):
        pltpu.sync_copy(zbuf, accsp.at[pl.ds(sid * NSL + q * CH, CH)])
    plsc.subcore_barrier()

    def _ea(i, c):
        r = i // (CH // L)
        cc = (i % (CH // L)) * L
        s16 = src_v[r, pl.ds(cc, L)]
        d16 = dst_v[r, pl.ds(cc, L)]
        e = plsc.load_gather(el_v, [s16]) + plsc.load_gather(er_v, [d16])
        e = jnp.maximum(e, 0.2 * e)
        x = jnp.exp(e - mvec)
        ex_v[pl.ds(i * L, L)] = x
        plsc.addupdate_scatter(den_v, [d16], x)
        return c
    lax.fori_loop(0, EPW // L, _ea, 0)

    pltpu.sync_copy(den_v, denpp_hbm.at[wid])

    cp = pltpu.async_copy(z_hbm.at[src_v.at[0]], rows0, gsem)
    for g in range(NCHUNK):
        buf = rows0 if g % 2 == 0 else rows1
        nbuf = rows1 if g % 2 == 0 else rows0
        cp.wait()
        if g + 1 < NCHUNK:
            cp = pltpu.async_copy(z_hbm.at[src_v.at[g + 1]], nbuf, gsem)

        def _scale(e, c, buf=buf, g=g):
            a = plsc.load_gather(ex_v, [jnp.full((L,), g * CH + e, jnp.int32)])
            for k in range(OUT // L):
                buf[e, pl.ds(k * L, L)] = buf[e, pl.ds(k * L, L)] * a
            return c
        lax.fori_loop(0, CH, _scale, 0)
        pltpu.sync_copy(buf, accsp.at[dst_v.at[g]], add=True)

    plsc.subcore_barrier()
    for q in range(NSL // CH):
        pltpu.sync_copy(accsp.at[pl.ds(sid * NSL + q * CH, CH)],
                        gatp_hbm.at[cid, pl.ds(sid * NSL + q * CH, CH)])


def _edge_phase(src3, dst3, el, er, m16, z):
    mesh = plsc.VectorSubcoreMesh(core_axis_name="c", subcore_axis_name="s")
    f = functools.partial(
        pl.kernel,
        out_type=[
            jax.ShapeDtypeStruct((NW, N), jnp.float32),
            jax.ShapeDtypeStruct((NC, N, OUT), jnp.float32),
        ],
        mesh=mesh,
        scratch_types=[
            pltpu.VMEM((EPW // CH, CH), jnp.int32),
            pltpu.VMEM((EPW // CH, CH), jnp.int32),
            pltpu.VMEM((N,), jnp.float32),
            pltpu.VMEM((N,), jnp.float32),
            pltpu.VMEM((EPW,), jnp.float32),
            pltpu.VMEM((N,), jnp.float32),
            pltpu.VMEM((L,), jnp.float32),
            pltpu.VMEM((CH, OUT), jnp.float32),
            pltpu.VMEM((CH, OUT), jnp.float32),
            pltpu.VMEM((CH, OUT), jnp.float32),
            pltpu.VMEM_SHARED((N, OUT), jnp.float32),
            pltpu.SemaphoreType.DMA,
        ],
        compiler_params=pltpu.CompilerParams(needs_layout_passes=False,
                                             use_tc_tiling_on_sc=False),
    )(_edge_body)
    return f(src3, dst3, el, er, m16, z)


def _gcn_body(adj_ref, sup_ref, b_ref, out_ref):
    k = pl.program_id(1)

    @pl.when(k == 0)
    def _init():
        out_ref[...] = jnp.zeros_like(out_ref)

    out_ref[...] += jnp.dot(adj_ref[...], sup_ref[...],
                            preferred_element_type=jnp.float32)

    @pl.when(k == pl.num_programs(1) - 1)
    def _fini():
        x = out_ref[...] + b_ref[...]
        out_ref[...] = jnp.where(x > 0, x, jnp.exp(x) - 1.0)


def _gcn(adj, support, b_gcn):
    return pl.pallas_call(
        _gcn_body,
        grid=(N // ROW_BLK, N // K_BLK),
        in_specs=[
            pl.BlockSpec((ROW_BLK, K_BLK), lambda i, k: (i, k)),
            pl.BlockSpec((K_BLK, OUT), lambda i, k: (k, 0)),
            pl.BlockSpec((1, OUT), lambda i, k: (0, 0)),
        ],
        out_specs=pl.BlockSpec((ROW_BLK, OUT), lambda i, k: (i, 0)),
        out_shape=jax.ShapeDtypeStruct((N, OUT), jnp.float32),
        compiler_params=pltpu.CompilerParams(
            dimension_semantics=("parallel", "arbitrary"),
        ),
    )(adj, support, b_gcn.reshape(1, OUT))


def _epi_body(gatp_ref, denpp_ref, b_ref, gcn_ref, out_ref):
    p = gatp_ref[0] + gatp_ref[1]
    d = jnp.sum(denpp_ref[...], axis=0)
    g = p / jnp.maximum(d, 1e-9)[:, None]
    g = g + b_ref[...]
    g = jnp.where(g > 0, g, jnp.exp(g) - 1.0)
    out_ref[...] = jnp.concatenate([g, gcn_ref[...]], axis=1)


def _epilogue(gatp, denpp, b_gat, gcn):
    return pl.pallas_call(
        _epi_body,
        grid=(N // ROW_BLK,),
        in_specs=[
            pl.BlockSpec((NC, ROW_BLK, OUT), lambda i: (0, i, 0)),
            pl.BlockSpec((NW, ROW_BLK), lambda i: (0, i)),
            pl.BlockSpec((1, OUT), lambda i: (0, 0)),
            pl.BlockSpec((ROW_BLK, OUT), lambda i: (i, 0)),
        ],
        out_specs=pl.BlockSpec((ROW_BLK, 2 * OUT), lambda i: (i, 0)),
        out_shape=jax.ShapeDtypeStruct((N, 2 * OUT), jnp.float32),
    )(gatp, denpp, b_gat.reshape(1, OUT), gcn)


def kernel(h, edge_index, adj, W_gat, attn_l, attn_r, b_gat, W_gcn, b_gcn):
    src3 = edge_index[0].reshape(NW, EPW // CH, CH)
    dst3 = edge_index[1].reshape(NW, EPW // CH, CH)

    z, el2, er2, support, m2 = _stage0(h, W_gat, attn_l, attn_r, W_gcn)
    denpp, gatp = _edge_phase(src3, dst3, el2.reshape(N), er2.reshape(N),
                              m2.reshape(L), z)
    gcn = _gcn(adj, support, b_gcn)
    return _epilogue(gatp, denpp, b_gat, gcn)
